# baseline (speedup 1.0000x reference)
import functools

import jax
import jax.numpy as jnp
from jax import lax
from jax.experimental import pallas as pl
from jax.experimental.pallas import tpu as pltpu
from jax.experimental.pallas import tpu_sc as plsc

VOCAB_SIZE = 100000
EMB_DIM = 128
BATCH = 4096
HIST_LEN = 50

NC = 2
NS = 16
NW = NC * NS

B_TOTAL = BATCH * HIST_LEN
B_PER_W = B_TOTAL // NW
CHUNK = 64
N_CHUNKS = B_PER_W // CHUNK
NBUF = 10
N_GROUPS = N_CHUNKS // NBUF
CPB = BATCH // CHUNK


def _make_gather():
  mesh = plsc.VectorSubcoreMesh(core_axis_name="c", subcore_axis_name="s",
                                num_cores=NC, num_subcores=NS)

  scratch = [pltpu.VMEM((B_PER_W,), jnp.int32)]
  scratch += [pltpu.VMEM((CHUNK, EMB_DIM), jnp.float32) for _ in range(NBUF)]
  scratch += [pltpu.SemaphoreType.DMA for _ in range(2 * NBUF)]

  @functools.partial(
      pl.kernel,
      out_type=jax.ShapeDtypeStruct((HIST_LEN, BATCH, EMB_DIM), jnp.float32),
      mesh=mesh,
      scratch_types=scratch,
      compiler_params=pltpu.CompilerParams(use_tc_tiling_on_sc=True,
                                           skip_device_barrier=True),
  )
  def gather_kernel(idx_hbm, table_hbm, out_hbm, idx_v, *bufs_and_sems):
    rows = bufs_and_sems[:NBUF]
    gsem = bufs_and_sems[NBUF:2 * NBUF]
    wsem = bufs_and_sems[2 * NBUF:]

    wid = lax.axis_index("s") * NC + lax.axis_index("c")
    base = pl.multiple_of(wid * B_PER_W, B_PER_W)
    u0 = pl.multiple_of(wid * N_CHUNKS, N_CHUNKS)
    pltpu.sync_copy(idx_hbm.at[pl.ds(base, B_PER_W)], idx_v)

    def unit(k):
      u = u0 + k
      return u // CPB, pl.multiple_of((u % CPB) * CHUNK, CHUNK)

    def gather(k, b):
      return pltpu.async_copy(
          table_hbm.at[idx_v.at[pl.ds(k * CHUNK, CHUNK)]], rows[b], gsem[b])

    for b in range(NBUF):
      gather(b, b)

    def group(i, _):
      for b in range(NBUF):
        k = i * NBUF + b
        h, boff = unit(k)
        pltpu.make_async_copy(
            table_hbm.at[idx_v.at[pl.ds(k * CHUNK, CHUNK)]], rows[b],
            gsem[b]).wait()
        pltpu.async_copy(rows[b], out_hbm.at[h, pl.ds(boff, CHUNK)], wsem[b])
        pltpu.make_async_copy(rows[b], out_hbm.at[h, pl.ds(boff, CHUNK)],
                              wsem[b]).wait()
        gather(k + NBUF, b)
      return 0

    lax.fori_loop(0, N_GROUPS - 1, group, 0)

    for b in range(NBUF):
      k = (N_GROUPS - 1) * NBUF + b
      h, boff = unit(k)
      pltpu.make_async_copy(
          table_hbm.at[idx_v.at[pl.ds(k * CHUNK, CHUNK)]], rows[b],
          gsem[b]).wait()
      pltpu.async_copy(rows[b], out_hbm.at[h, pl.ds(boff, CHUNK)], wsem[b])
    for b in range(NBUF):
      k = (N_GROUPS - 1) * NBUF + b
      h, boff = unit(k)
      pltpu.make_async_copy(rows[b], out_hbm.at[h, pl.ds(boff, CHUNK)],
                            wsem[b]).wait()

  return gather_kernel


_gather = _make_gather()


def kernel(x, table):
  idx = x.astype(jnp.int32).T.reshape(-1)
  out = _gather(idx, table)
  return out.transpose(1, 0, 2)
